# Initial kernel scaffold; baseline (speedup 1.0000x reference)
import jax
import jax.numpy as jnp
from jax.experimental import pallas as pl


def kernel(x, edge_index, edge_weight, W1, b1, W2, b2, W3, b3, W4, b4):
    raise NotImplementedError("write your pallas kernel here")


# speedup vs baseline: 24.2037x; 24.2037x over previous
import functools

import jax
import jax.numpy as jnp
from jax import lax
from jax.experimental import pallas as pl
from jax.experimental.pallas import tpu as pltpu
from jax.experimental.pallas import tpu_sc as plsc

N = 100000
E = 1600000
F_IN = 128
H = 32
C = 10

NC = 2
NS = 16

NPAD = 100352
EPAD = 1638400

ROWS_PER_TILE = NPAD // NS
ZCH = 392

DEG_VECS_PER_W = EPAD // (NC * NS) // 16
DEG_CHUNK = 400
DEG_STAGES = DEG_VECS_PER_W // DEG_CHUNK

MSG_BLOCKS_PER_TILE = (EPAD // 128) // NS
MSG_CHUNK = 50
MSG_STAGES = MSG_BLOCKS_PER_TILE // MSG_CHUNK

_mesh = plsc.VectorSubcoreMesh(core_axis_name="c", subcore_axis_name="s")


@functools.partial(
    pl.kernel,
    out_type=jax.ShapeDtypeStruct((NC * NS, NPAD), jnp.float32),
    mesh=_mesh,
    compiler_params=pltpu.CompilerParams(
        needs_layout_passes=False, use_tc_tiling_on_sc=False),
    scratch_types=[
        pltpu.VMEM((NPAD,), jnp.float32),
        pltpu.VMEM((DEG_CHUNK, 16), jnp.int32),
    ],
)
def _sc_degree(dst_hbm, out_hbm, deg_v, idx_v):
    cid = lax.axis_index("c")
    sid = lax.axis_index("s")
    wid = sid * NC + cid
    zero16 = jnp.zeros((16,), jnp.float32)
    ones16 = jnp.ones((16,), jnp.float32)

    def _zero(i, c):
        deg_v[pl.ds(i * 16, 16)] = zero16
        return c

    lax.fori_loop(0, NPAD // 16, _zero, 0)

    base = wid * DEG_VECS_PER_W

    def _stage(s, c):
        pltpu.sync_copy(dst_hbm.at[pl.ds(base + s * DEG_CHUNK, DEG_CHUNK)], idx_v)

        def _vec(j, c2):
            idx = idx_v[j, :]
            plsc.addupdate_scatter(deg_v, [idx], ones16)
            return c2

        lax.fori_loop(0, DEG_CHUNK, _vec, 0)
        return c

    lax.fori_loop(0, DEG_STAGES, _stage, 0)
    pltpu.sync_copy(deg_v, out_hbm.at[wid])


@functools.partial(
    pl.kernel,
    out_type=(
        jax.ShapeDtypeStruct((NPAD, 16), jnp.float32),
        jax.ShapeDtypeStruct((NPAD, 16), jnp.float32),
    ),
    mesh=_mesh,
    compiler_params=pltpu.CompilerParams(
        needs_layout_passes=False, use_tc_tiling_on_sc=False),
    scratch_types=[
        pltpu.VMEM_SHARED((NPAD, 16), jnp.float32),
        pltpu.VMEM((MSG_CHUNK, 128), jnp.int32),
        pltpu.VMEM((MSG_CHUNK, 128), jnp.int32),
        pltpu.VMEM((128, 16), jnp.float32),
        pltpu.VMEM((ZCH, 16), jnp.float32),
        pltpu.SemaphoreType.DMA,
    ],
)
def _sc_message(src_hbm, dst_hbm, y0_hbm, y1_hbm, out0_hbm, out1_hbm,
                acc, src_v, dst_v, rows, zbuf, sem):
    cid = lax.axis_index("c")
    sid = lax.axis_index("s")

    def _run(y_hbm, out_hbm):
        zero16 = jnp.zeros((16,), jnp.float32)

        def _zb(i, c):
            zbuf[i, :] = zero16
            return c

        lax.fori_loop(0, ZCH, _zb, 0)
        row0 = sid * ROWS_PER_TILE

        def _za(k, c):
            pltpu.sync_copy(zbuf, acc.at[pl.ds(row0 + k * ZCH, ZCH)])
            return c

        lax.fori_loop(0, ROWS_PER_TILE // ZCH, _za, 0)
        plsc.subcore_barrier()

        blk0 = sid * MSG_BLOCKS_PER_TILE

        def _stage(s, c):
            r0 = blk0 + s * MSG_CHUNK
            pltpu.sync_copy(src_hbm.at[pl.ds(r0, MSG_CHUNK)], src_v)
            pltpu.sync_copy(dst_hbm.at[pl.ds(r0, MSG_CHUNK)], dst_v)

            def _blk(j, c2):
                pltpu.async_copy(y_hbm.at[src_v.at[j]], rows, sem).wait()
                pltpu.sync_copy(rows, acc.at[dst_v.at[j]], add=True)
                return c2

            lax.fori_loop(0, MSG_CHUNK, _blk, 0)
            return c

        lax.fori_loop(0, MSG_STAGES, _stage, 0)
        plsc.subcore_barrier()
        pltpu.sync_copy(acc.at[pl.ds(row0, ROWS_PER_TILE)],
                        out_hbm.at[pl.ds(row0, ROWS_PER_TILE)])

    @pl.when(cid == 0)
    def _():
        _run(y0_hbm, out0_hbm)

    @pl.when(cid == 1)
    def _():
        _run(y1_hbm, out1_hbm)


_BLK = 1024
_GRID = NPAD // _BLK


def _prep_body(x_ref, w1_ref, degp_ref, y0_ref, y1_ref, dinv_ref):
    deg = jnp.sum(degp_ref[...], axis=0) + 1.0
    dinv = lax.rsqrt(deg)
    xw = jnp.dot(x_ref[...], w1_ref[...], preferred_element_type=jnp.float32)
    y = xw * dinv[:, None]
    y0_ref[...] = y[:, :16]
    y1_ref[...] = y[:, 16:]
    dinv_ref[...] = dinv[:, None]


_prep = pl.pallas_call(
    _prep_body,
    grid=(_GRID,),
    in_specs=[
        pl.BlockSpec((_BLK, F_IN), lambda i: (i, 0)),
        pl.BlockSpec((F_IN, H), lambda i: (0, 0)),
        pl.BlockSpec((NC * NS, _BLK), lambda i: (0, i)),
    ],
    out_specs=[
        pl.BlockSpec((_BLK, 16), lambda i: (i, 0)),
        pl.BlockSpec((_BLK, 16), lambda i: (i, 0)),
        pl.BlockSpec((_BLK, 1), lambda i: (i, 0)),
    ],
    out_shape=[
        jax.ShapeDtypeStruct((NPAD, 16), jnp.float32),
        jax.ShapeDtypeStruct((NPAD, 16), jnp.float32),
        jax.ShapeDtypeStruct((NPAD, 1), jnp.float32),
    ],
)


def _head_body(acc0_ref, acc1_ref, y0_ref, y1_ref, dinv_ref,
               b1_ref, w2_ref, b2_ref, w3_ref, b3_ref, w4_ref, b4_ref,
               out_ref):
    fsum = jnp.concatenate(
        [acc0_ref[...] + y0_ref[...], acc1_ref[...] + y1_ref[...]], axis=1)
    h = jnp.maximum(fsum * dinv_ref[...] + b1_ref[...], 0.0)
    h = jnp.maximum(
        jnp.dot(h, w2_ref[...], preferred_element_type=jnp.float32)
        + b2_ref[...], 0.0)
    h = jnp.maximum(
        jnp.dot(h, w3_ref[...], preferred_element_type=jnp.float32)
        + b3_ref[...], 0.0)
    logits = (jnp.dot(h, w4_ref[...], preferred_element_type=jnp.float32)
              + b4_ref[...])
    m = jnp.max(logits, axis=1, keepdims=True)
    lse = jnp.log(jnp.sum(jnp.exp(logits - m), axis=1, keepdims=True)) + m
    out_ref[...] = logits - lse


_head = pl.pallas_call(
    _head_body,
    grid=(_GRID,),
    in_specs=[
        pl.BlockSpec((_BLK, 16), lambda i: (i, 0)),
        pl.BlockSpec((_BLK, 16), lambda i: (i, 0)),
        pl.BlockSpec((_BLK, 16), lambda i: (i, 0)),
        pl.BlockSpec((_BLK, 16), lambda i: (i, 0)),
        pl.BlockSpec((_BLK, 1), lambda i: (i, 0)),
        pl.BlockSpec((1, H), lambda i: (0, 0)),
        pl.BlockSpec((H, H // 2), lambda i: (0, 0)),
        pl.BlockSpec((1, H // 2), lambda i: (0, 0)),
        pl.BlockSpec((H // 2, H // 4), lambda i: (0, 0)),
        pl.BlockSpec((1, H // 4), lambda i: (0, 0)),
        pl.BlockSpec((H // 4, C), lambda i: (0, 0)),
        pl.BlockSpec((1, C), lambda i: (0, 0)),
    ],
    out_specs=pl.BlockSpec((_BLK, C), lambda i: (i, 0)),
    out_shape=jax.ShapeDtypeStruct((NPAD, C), jnp.float32),
)


def kernel(x, edge_index, edge_weight, W1, b1, W2, b2, W3, b3, W4, b4):
    del edge_weight
    src = edge_index[0]
    dst = edge_index[1]
    pad = jnp.full((EPAD - E,), N, dtype=jnp.int32)
    srcp = jnp.concatenate([src, pad]).reshape(EPAD // 128, 128)
    dstp = jnp.concatenate([dst, pad]).reshape(EPAD // 128, 128)

    degp = _sc_degree(dstp.reshape(EPAD // 16, 16))
    y0, y1, dinv = _prep(x, W1, degp)
    acc0, acc1 = _sc_message(srcp, dstp, y0, y1)
    out = _head(acc0, acc1, y0, y1, dinv,
                b1.reshape(1, H), W2, b2.reshape(1, H // 2),
                W3, b3.reshape(1, H // 4), W4, b4.reshape(1, C))
    return out[:N]


# speedup vs baseline: 34.6651x; 1.4322x over previous
import functools

import jax
import jax.numpy as jnp
from jax import lax
from jax.experimental import pallas as pl
from jax.experimental.pallas import tpu as pltpu
from jax.experimental.pallas import tpu_sc as plsc

N = 100000
E = 1600000
F_IN = 128
H = 32
C = 10

NC = 2
NS = 16

NPAD = 100352
EPAD = 1638400

ROWS_PER_TILE = NPAD // NS
ZCH = 392

DEG_VECS_PER_W = EPAD // (NC * NS) // 16
DEG_CHUNK = 400
DEG_STAGES = DEG_VECS_PER_W // DEG_CHUNK

MSG_BLOCKS_PER_TILE = (EPAD // 128) // NS
MSG_CHUNK = 50
MSG_STAGES = MSG_BLOCKS_PER_TILE // MSG_CHUNK
NBUF = 4
PIPE = 2

_mesh = plsc.VectorSubcoreMesh(core_axis_name="c", subcore_axis_name="s")


@functools.partial(
    pl.kernel,
    out_type=jax.ShapeDtypeStruct((NC * NS, NPAD), jnp.float32),
    mesh=_mesh,
    compiler_params=pltpu.CompilerParams(
        needs_layout_passes=False, use_tc_tiling_on_sc=False),
    scratch_types=[
        pltpu.VMEM((NPAD,), jnp.float32),
        pltpu.VMEM((DEG_CHUNK, 16), jnp.int32),
    ],
)
def _sc_degree(dst_hbm, out_hbm, deg_v, idx_v):
    cid = lax.axis_index("c")
    sid = lax.axis_index("s")
    wid = sid * NC + cid
    zero16 = jnp.zeros((16,), jnp.float32)
    ones16 = jnp.ones((16,), jnp.float32)

    def _zero(i, c):
        deg_v[pl.ds(i * 16, 16)] = zero16
        return c

    lax.fori_loop(0, NPAD // 16, _zero, 0, unroll=8)

    base = wid * DEG_VECS_PER_W

    def _stage(s, c):
        pltpu.sync_copy(dst_hbm.at[pl.ds(base + s * DEG_CHUNK, DEG_CHUNK)], idx_v)

        def _vec(j, c2):
            idx = idx_v[j, :]
            plsc.addupdate_scatter(deg_v, [idx], ones16)
            return c2

        lax.fori_loop(0, DEG_CHUNK, _vec, 0, unroll=8)
        return c

    lax.fori_loop(0, DEG_STAGES, _stage, 0)
    pltpu.sync_copy(deg_v, out_hbm.at[wid])


@functools.partial(
    pl.kernel,
    out_type=(
        jax.ShapeDtypeStruct((NPAD, 16), jnp.float32),
        jax.ShapeDtypeStruct((NPAD, 16), jnp.float32),
    ),
    mesh=_mesh,
    compiler_params=pltpu.CompilerParams(
        needs_layout_passes=False, use_tc_tiling_on_sc=False),
    scratch_types=[
        pltpu.VMEM_SHARED((NPAD, 16), jnp.float32),
        pltpu.VMEM((MSG_CHUNK, 128), jnp.int32),
        pltpu.VMEM((MSG_CHUNK, 128), jnp.int32),
        pltpu.VMEM((NBUF, 128, 16), jnp.float32),
        pltpu.VMEM((ZCH, 16), jnp.float32),
        pltpu.SemaphoreType.DMA,
        pltpu.SemaphoreType.DMA,
        pltpu.SemaphoreType.DMA,
        pltpu.SemaphoreType.DMA,
        pltpu.SemaphoreType.DMA,
        pltpu.SemaphoreType.DMA,
        pltpu.SemaphoreType.DMA,
        pltpu.SemaphoreType.DMA,
    ],
)
def _sc_message(src_hbm, dst_hbm, y0_hbm, y1_hbm, out0_hbm, out1_hbm,
                acc, src_v, dst_v, rows, zbuf,
                gsem0, gsem1, gsem2, gsem3, ssem0, ssem1, ssem2, ssem3):
    cid = lax.axis_index("c")
    sid = lax.axis_index("s")
    gsems = (gsem0, gsem1, gsem2, gsem3)
    ssems = (ssem0, ssem1, ssem2, ssem3)

    def _run(y_hbm, out_hbm):
        zero16 = jnp.zeros((16,), jnp.float32)

        def _zb(i, c):
            zbuf[i, :] = zero16
            return c

        lax.fori_loop(0, ZCH, _zb, 0, unroll=8)
        row0 = sid * ROWS_PER_TILE

        def _za(k, c):
            pltpu.sync_copy(zbuf, acc.at[pl.ds(row0 + k * ZCH, ZCH)])
            return c

        lax.fori_loop(0, ROWS_PER_TILE // ZCH, _za, 0)
        plsc.subcore_barrier()

        blk0 = sid * MSG_BLOCKS_PER_TILE

        def _chunk(s, c):
            r0 = blk0 + s * MSG_CHUNK
            pltpu.sync_copy(src_hbm.at[pl.ds(r0, MSG_CHUNK)], src_v)
            pltpu.sync_copy(dst_hbm.at[pl.ds(r0, MSG_CHUNK)], dst_v)
            gd, sd = {}, {}
            for j in range(PIPE):
                gd[j % NBUF] = pltpu.async_copy(
                    y_hbm.at[src_v.at[j]], rows.at[j % NBUF], gsems[j % NBUF])
            for j in range(MSG_CHUNK):
                g = j + PIPE
                if g < MSG_CHUNK:
                    bg = g % NBUF
                    if g >= NBUF:
                        sd[bg].wait()
                    gd[bg] = pltpu.async_copy(
                        y_hbm.at[src_v.at[g]], rows.at[bg], gsems[bg])
                b = j % NBUF
                gd[b].wait()
                sd[b] = pltpu.async_copy(
                    rows.at[b], acc.at[dst_v.at[j]], ssems[b], add=True)
            for b in range(NBUF):
                sd[b].wait()
            return c

        lax.fori_loop(0, MSG_STAGES, _chunk, 0)
        plsc.subcore_barrier()
        pltpu.sync_copy(acc.at[pl.ds(row0, ROWS_PER_TILE)],
                        out_hbm.at[pl.ds(row0, ROWS_PER_TILE)])

    @pl.when(cid == 0)
    def _():
        _run(y0_hbm, out0_hbm)

    @pl.when(cid == 1)
    def _():
        _run(y1_hbm, out1_hbm)


_BLK = 1024
_GRID = NPAD // _BLK


def _prep_body(x_ref, w1_ref, degp_ref, y0_ref, y1_ref, dinv_ref):
    deg = jnp.sum(degp_ref[...], axis=0) + 1.0
    dinv = lax.rsqrt(deg)
    xw = jnp.dot(x_ref[...], w1_ref[...], preferred_element_type=jnp.float32)
    y = xw * dinv[:, None]
    y0_ref[...] = y[:, :16]
    y1_ref[...] = y[:, 16:]
    dinv_ref[...] = dinv[:, None]


_prep = pl.pallas_call(
    _prep_body,
    grid=(_GRID,),
    in_specs=[
        pl.BlockSpec((_BLK, F_IN), lambda i: (i, 0)),
        pl.BlockSpec((F_IN, H), lambda i: (0, 0)),
        pl.BlockSpec((NC * NS, _BLK), lambda i: (0, i)),
    ],
    out_specs=[
        pl.BlockSpec((_BLK, 16), lambda i: (i, 0)),
        pl.BlockSpec((_BLK, 16), lambda i: (i, 0)),
        pl.BlockSpec((_BLK, 1), lambda i: (i, 0)),
    ],
    out_shape=[
        jax.ShapeDtypeStruct((NPAD, 16), jnp.float32),
        jax.ShapeDtypeStruct((NPAD, 16), jnp.float32),
        jax.ShapeDtypeStruct((NPAD, 1), jnp.float32),
    ],
)


def _head_body(acc0_ref, acc1_ref, y0_ref, y1_ref, dinv_ref,
               b1_ref, w2_ref, b2_ref, w3_ref, b3_ref, w4_ref, b4_ref,
               out_ref):
    fsum = jnp.concatenate(
        [acc0_ref[...] + y0_ref[...], acc1_ref[...] + y1_ref[...]], axis=1)
    h = jnp.maximum(fsum * dinv_ref[...] + b1_ref[...], 0.0)
    h = jnp.maximum(
        jnp.dot(h, w2_ref[...], preferred_element_type=jnp.float32)
        + b2_ref[...], 0.0)
    h = jnp.maximum(
        jnp.dot(h, w3_ref[...], preferred_element_type=jnp.float32)
        + b3_ref[...], 0.0)
    logits = (jnp.dot(h, w4_ref[...], preferred_element_type=jnp.float32)
              + b4_ref[...])
    m = jnp.max(logits, axis=1, keepdims=True)
    lse = jnp.log(jnp.sum(jnp.exp(logits - m), axis=1, keepdims=True)) + m
    out_ref[...] = logits - lse


_head = pl.pallas_call(
    _head_body,
    grid=(_GRID,),
    in_specs=[
        pl.BlockSpec((_BLK, 16), lambda i: (i, 0)),
        pl.BlockSpec((_BLK, 16), lambda i: (i, 0)),
        pl.BlockSpec((_BLK, 16), lambda i: (i, 0)),
        pl.BlockSpec((_BLK, 16), lambda i: (i, 0)),
        pl.BlockSpec((_BLK, 1), lambda i: (i, 0)),
        pl.BlockSpec((1, H), lambda i: (0, 0)),
        pl.BlockSpec((H, H // 2), lambda i: (0, 0)),
        pl.BlockSpec((1, H // 2), lambda i: (0, 0)),
        pl.BlockSpec((H // 2, H // 4), lambda i: (0, 0)),
        pl.BlockSpec((1, H // 4), lambda i: (0, 0)),
        pl.BlockSpec((H // 4, C), lambda i: (0, 0)),
        pl.BlockSpec((1, C), lambda i: (0, 0)),
    ],
    out_specs=pl.BlockSpec((_BLK, C), lambda i: (i, 0)),
    out_shape=jax.ShapeDtypeStruct((NPAD, C), jnp.float32),
)


def kernel(x, edge_index, edge_weight, W1, b1, W2, b2, W3, b3, W4, b4):
    del edge_weight
    src = edge_index[0]
    dst = edge_index[1]
    pad = jnp.full((EPAD - E,), N, dtype=jnp.int32)
    srcp = jnp.concatenate([src, pad]).reshape(EPAD // 128, 128)
    dstp = jnp.concatenate([dst, pad]).reshape(EPAD // 128, 128)

    degp = _sc_degree(dstp.reshape(EPAD // 16, 16))
    y0, y1, dinv = _prep(x, W1, degp)
    acc0, acc1 = _sc_message(srcp, dstp, y0, y1)
    out = _head(acc0, acc1, y0, y1, dinv,
                b1.reshape(1, H), W2, b2.reshape(1, H // 2),
                W3, b3.reshape(1, H // 4), W4, b4.reshape(1, C))
    return out[:N]


# speedup vs baseline: 42.1896x; 1.2171x over previous
import functools

import jax
import jax.numpy as jnp
from jax import lax
from jax.experimental import pallas as pl
from jax.experimental.pallas import tpu as pltpu
from jax.experimental.pallas import tpu_sc as plsc

N = 100000
E = 1600000
F_IN = 128
H = 32
C = 10

NC = 2
NS = 16

NPAD = 100352
EPAD = 1703936
EROWS = EPAD // 128

ROWS_PER_TILE = NPAD // NS
ZCH = 224

DEG_ROWS_PER_W = EROWS // (NC * NS)
DEG_CHUNK = 52
DEG_STAGES = DEG_ROWS_PER_W // DEG_CHUNK

MSG_BLOCKS_PER_TILE = EROWS // NS
MSG_CHUNK = 52
MSG_STAGES = MSG_BLOCKS_PER_TILE // MSG_CHUNK
NBUF = 6
PIPE = 3

_mesh = plsc.VectorSubcoreMesh(core_axis_name="c", subcore_axis_name="s")


@functools.partial(
    pl.kernel,
    out_type=jax.ShapeDtypeStruct((NC * NS, NPAD), jnp.float32),
    mesh=_mesh,
    compiler_params=pltpu.CompilerParams(
        needs_layout_passes=False, use_tc_tiling_on_sc=False),
    scratch_types=[
        pltpu.VMEM((NPAD,), jnp.float32),
        pltpu.VMEM((DEG_CHUNK, 128), jnp.int32),
    ],
)
def _sc_degree(dst_hbm, out_hbm, deg_v, idx_v):
    cid = lax.axis_index("c")
    sid = lax.axis_index("s")
    wid = sid * NC + cid
    zero16 = jnp.zeros((16,), jnp.float32)
    ones16 = jnp.ones((16,), jnp.float32)

    def _zero(i, c):
        deg_v[pl.ds(i * 16, 16)] = zero16
        return c

    lax.fori_loop(0, NPAD // 16, _zero, 0, unroll=8)

    base = wid * DEG_ROWS_PER_W

    def _stage(s, c):
        pltpu.sync_copy(dst_hbm.at[pl.ds(base + s * DEG_CHUNK, DEG_CHUNK)],
                        idx_v)

        def _row(j, c2):
            for k in range(8):
                idx = idx_v[j, pl.ds(k * 16, 16)]
                plsc.addupdate_scatter(deg_v, [idx], ones16)
            return c2

        lax.fori_loop(0, DEG_CHUNK, _row, 0, unroll=2)
        return c

    lax.fori_loop(0, DEG_STAGES, _stage, 0)
    pltpu.sync_copy(deg_v, out_hbm.at[wid])


@functools.partial(
    pl.kernel,
    out_type=(
        jax.ShapeDtypeStruct((NPAD, 16), jnp.float32),
        jax.ShapeDtypeStruct((NPAD, 16), jnp.float32),
    ),
    mesh=_mesh,
    compiler_params=pltpu.CompilerParams(
        needs_layout_passes=False, use_tc_tiling_on_sc=False),
    scratch_types=[
        pltpu.VMEM_SHARED((NPAD, 16), jnp.float32),
        pltpu.VMEM((MSG_CHUNK, 128), jnp.int32),
        pltpu.VMEM((MSG_CHUNK, 128), jnp.int32),
        pltpu.VMEM((NBUF, 128, 16), jnp.float32),
        pltpu.VMEM((ZCH, 16), jnp.float32),
        pltpu.SemaphoreType.DMA,
        pltpu.SemaphoreType.DMA,
        pltpu.SemaphoreType.DMA,
        pltpu.SemaphoreType.DMA,
        pltpu.SemaphoreType.DMA,
        pltpu.SemaphoreType.DMA,
        pltpu.SemaphoreType.DMA,
        pltpu.SemaphoreType.DMA,
        pltpu.SemaphoreType.DMA,
        pltpu.SemaphoreType.DMA,
        pltpu.SemaphoreType.DMA,
        pltpu.SemaphoreType.DMA,
    ],
)
def _sc_message(src_hbm, dst_hbm, y0_hbm, y1_hbm, out0_hbm, out1_hbm,
                acc, src_v, dst_v, rows, zbuf,
                gsem0, gsem1, gsem2, gsem3, gsem4, gsem5,
                ssem0, ssem1, ssem2, ssem3, ssem4, ssem5):
    cid = lax.axis_index("c")
    sid = lax.axis_index("s")
    gsems = (gsem0, gsem1, gsem2, gsem3, gsem4, gsem5)
    ssems = (ssem0, ssem1, ssem2, ssem3, ssem4, ssem5)

    def _run(y_hbm, out_hbm):
        zero16 = jnp.zeros((16,), jnp.float32)

        def _zb(i, c):
            zbuf[i, :] = zero16
            return c

        lax.fori_loop(0, ZCH, _zb, 0, unroll=8)
        row0 = sid * ROWS_PER_TILE

        def _za(k, c):
            pltpu.sync_copy(zbuf, acc.at[pl.ds(row0 + k * ZCH, ZCH)])
            return c

        lax.fori_loop(0, ROWS_PER_TILE // ZCH, _za, 0)
        plsc.subcore_barrier()

        blk0 = sid * MSG_BLOCKS_PER_TILE

        def _chunk(s, c):
            r0 = blk0 + s * MSG_CHUNK
            pltpu.sync_copy(src_hbm.at[pl.ds(r0, MSG_CHUNK)], src_v)
            pltpu.sync_copy(dst_hbm.at[pl.ds(r0, MSG_CHUNK)], dst_v)
            gd, sd = {}, {}
            for j in range(PIPE):
                gd[j % NBUF] = pltpu.async_copy(
                    y_hbm.at[src_v.at[j]], rows.at[j % NBUF], gsems[j % NBUF])
            for j in range(MSG_CHUNK):
                g = j + PIPE
                if g < MSG_CHUNK:
                    bg = g % NBUF
                    if g >= NBUF:
                        sd[bg].wait()
                    gd[bg] = pltpu.async_copy(
                        y_hbm.at[src_v.at[g]], rows.at[bg], gsems[bg])
                b = j % NBUF
                gd[b].wait()
                sd[b] = pltpu.async_copy(
                    rows.at[b], acc.at[dst_v.at[j]], ssems[b], add=True)
            for b in range(NBUF):
                sd[b].wait()
            return c

        lax.fori_loop(0, MSG_STAGES, _chunk, 0)
        plsc.subcore_barrier()
        pltpu.sync_copy(acc.at[pl.ds(row0, ROWS_PER_TILE)],
                        out_hbm.at[pl.ds(row0, ROWS_PER_TILE)])

    @pl.when(cid == 0)
    def _():
        _run(y0_hbm, out0_hbm)

    @pl.when(cid == 1)
    def _():
        _run(y1_hbm, out1_hbm)


_BLK = 1024
_GRID = NPAD // _BLK


def _prep_body(x_ref, w1_ref, degp_ref, y0_ref, y1_ref, dinv_ref):
    deg = jnp.sum(degp_ref[...], axis=0)
    dinv = lax.rsqrt(deg)
    xw = jnp.dot(x_ref[...], w1_ref[...], preferred_element_type=jnp.float32)
    y = xw * dinv[:, None]
    y0_ref[...] = y[:, :16]
    y1_ref[...] = y[:, 16:]
    dinv_ref[...] = dinv[:, None]


_prep = pl.pallas_call(
    _prep_body,
    grid=(_GRID,),
    in_specs=[
        pl.BlockSpec((_BLK, F_IN), lambda i: (i, 0)),
        pl.BlockSpec((F_IN, H), lambda i: (0, 0)),
        pl.BlockSpec((NC * NS, _BLK), lambda i: (0, i)),
    ],
    out_specs=[
        pl.BlockSpec((_BLK, 16), lambda i: (i, 0)),
        pl.BlockSpec((_BLK, 16), lambda i: (i, 0)),
        pl.BlockSpec((_BLK, 1), lambda i: (i, 0)),
    ],
    out_shape=[
        jax.ShapeDtypeStruct((NPAD, 16), jnp.float32),
        jax.ShapeDtypeStruct((NPAD, 16), jnp.float32),
        jax.ShapeDtypeStruct((NPAD, 1), jnp.float32),
    ],
)


def _head_body(acc0_ref, acc1_ref, dinv_ref,
               b1_ref, w2_ref, b2_ref, w3_ref, b3_ref, w4_ref, b4_ref,
               out_ref):
    acc = jnp.concatenate([acc0_ref[...], acc1_ref[...]], axis=1)
    h = jnp.maximum(acc * dinv_ref[...] + b1_ref[...], 0.0)
    h = jnp.maximum(
        jnp.dot(h, w2_ref[...], preferred_element_type=jnp.float32)
        + b2_ref[...], 0.0)
    h = jnp.maximum(
        jnp.dot(h, w3_ref[...], preferred_element_type=jnp.float32)
        + b3_ref[...], 0.0)
    logits = (jnp.dot(h, w4_ref[...], preferred_element_type=jnp.float32)
              + b4_ref[...])
    m = jnp.max(logits, axis=1, keepdims=True)
    lse = jnp.log(jnp.sum(jnp.exp(logits - m), axis=1, keepdims=True)) + m
    out_ref[...] = logits - lse


_head = pl.pallas_call(
    _head_body,
    grid=(_GRID,),
    in_specs=[
        pl.BlockSpec((_BLK, 16), lambda i: (i, 0)),
        pl.BlockSpec((_BLK, 16), lambda i: (i, 0)),
        pl.BlockSpec((_BLK, 1), lambda i: (i, 0)),
        pl.BlockSpec((1, H), lambda i: (0, 0)),
        pl.BlockSpec((H, H // 2), lambda i: (0, 0)),
        pl.BlockSpec((1, H // 2), lambda i: (0, 0)),
        pl.BlockSpec((H // 2, H // 4), lambda i: (0, 0)),
        pl.BlockSpec((1, H // 4), lambda i: (0, 0)),
        pl.BlockSpec((H // 4, C), lambda i: (0, 0)),
        pl.BlockSpec((1, C), lambda i: (0, 0)),
    ],
    out_specs=pl.BlockSpec((_BLK, C), lambda i: (i, 0)),
    out_shape=jax.ShapeDtypeStruct((N, C), jnp.float32),
)


def kernel(x, edge_index, edge_weight, W1, b1, W2, b2, W3, b3, W4, b4):
    del edge_weight
    src = edge_index[0]
    dst = edge_index[1]
    loop = jnp.arange(N, dtype=jnp.int32)
    fill = jnp.full((EPAD - E - N,), N, dtype=jnp.int32)
    srcp = jnp.concatenate([src, loop, fill]).reshape(EROWS, 128)
    dstp = jnp.concatenate([dst, loop, fill]).reshape(EROWS, 128)

    degp = _sc_degree(dstp)
    y0, y1, dinv = _prep(x, W1, degp)
    acc0, acc1 = _sc_message(srcp, dstp, y0, y1)
    return _head(acc0, acc1, dinv,
                 b1.reshape(1, H), W2, b2.reshape(1, H // 2),
                 W3, b3.reshape(1, H // 4), W4, b4.reshape(1, C))


# speedup vs baseline: 51.0381x; 1.2097x over previous
import functools

import jax
import jax.numpy as jnp
from jax import lax
from jax.experimental import pallas as pl
from jax.experimental.pallas import tpu as pltpu
from jax.experimental.pallas import tpu_sc as plsc

N = 100000
E = 1600000
F_IN = 128
H = 32
C = 10

NC = 2
NS = 16

NPAD = 100352
ESELF = E + NPAD
EPAD = 1703936
EROWS = EPAD // 128
ER_IN = E // 128

ROWS_PER_TILE = NPAD // NS
ZCH = 224

DEG_ROWS_PER_W = EROWS // (NC * NS)
DEG_CHUNK = 52
DEG_STAGES = DEG_ROWS_PER_W // DEG_CHUNK

MSG_BLOCKS_PER_TILE = EROWS // NS
MSG_CHUNK = 52
MSG_STAGES = MSG_BLOCKS_PER_TILE // MSG_CHUNK
NBUF = 6
PIPE = 3

_mesh = plsc.VectorSubcoreMesh(core_axis_name="c", subcore_axis_name="s")


def _edges_body(src_in, dst_in, srcp_ref, dstp_ref):
    i = pl.program_id(0)
    row = i * 128 + lax.broadcasted_iota(jnp.int32, (128, 128), 0)
    lane = lax.broadcasted_iota(jnp.int32, (128, 128), 1)
    self_id = (row - ER_IN) * 128 + lane
    tail = jnp.where(row < ER_IN + NPAD // 128, self_id, N)
    srcp_ref[...] = jnp.where(row < ER_IN, src_in[0], tail) * 8
    dstp_ref[...] = jnp.where(row < ER_IN, dst_in[0], tail)


_edges = pl.pallas_call(
    _edges_body,
    grid=(EROWS // 128,),
    in_specs=[
        pl.BlockSpec((1, 128, 128),
                     lambda i: (0, jnp.minimum(i, ER_IN // 128), 0)),
        pl.BlockSpec((1, 128, 128),
                     lambda i: (1, jnp.minimum(i, ER_IN // 128), 0)),
    ],
    out_specs=[
        pl.BlockSpec((128, 128), lambda i: (i, 0)),
        pl.BlockSpec((128, 128), lambda i: (i, 0)),
    ],
    out_shape=[
        jax.ShapeDtypeStruct((EROWS, 128), jnp.int32),
        jax.ShapeDtypeStruct((EROWS, 128), jnp.int32),
    ],
)


@functools.partial(
    pl.kernel,
    out_type=tuple(
        jax.ShapeDtypeStruct((8, NPAD), jnp.float32) for _ in range(4)),
    mesh=_mesh,
    compiler_params=pltpu.CompilerParams(
        needs_layout_passes=False, use_tc_tiling_on_sc=False),
    scratch_types=[
        pltpu.VMEM((NPAD,), jnp.float32),
        pltpu.VMEM((DEG_CHUNK, 128), jnp.int32),
    ],
)
def _sc_degree(dst_hbm, out0, out1, out2, out3, deg_v, idx_v):
    cid = lax.axis_index("c")
    sid = lax.axis_index("s")
    wid = sid * NC + cid
    zero16 = jnp.zeros((16,), jnp.float32)
    ones16 = jnp.ones((16,), jnp.float32)

    def _zero(i, c):
        deg_v[pl.ds(i * 16, 16)] = zero16
        return c

    lax.fori_loop(0, NPAD // 16, _zero, 0, unroll=8)

    base = wid * DEG_ROWS_PER_W

    def _stage(s, c):
        pltpu.sync_copy(dst_hbm.at[pl.ds(base + s * DEG_CHUNK, DEG_CHUNK)],
                        idx_v)

        def _row(j, c2):
            for k in range(8):
                idx = idx_v[j, pl.ds(k * 16, 16)]
                plsc.addupdate_scatter(deg_v, [idx], ones16)
            return c2

        lax.fori_loop(0, DEG_CHUNK, _row, 0, unroll=2)
        return c

    lax.fori_loop(0, DEG_STAGES, _stage, 0)
    for k, out_k in enumerate((out0, out1, out2, out3)):
        @pl.when(wid // 8 == k)
        def _(out_k=out_k):
            pltpu.sync_copy(deg_v, out_k.at[wid % 8])


@functools.partial(
    pl.kernel,
    out_type=(
        jax.ShapeDtypeStruct((NPAD, 128), jnp.float32),
        jax.ShapeDtypeStruct((NPAD, 128), jnp.float32),
    ),
    mesh=_mesh,
    compiler_params=pltpu.CompilerParams(
        needs_layout_passes=False, use_tc_tiling_on_sc=False),
    scratch_types=[
        pltpu.VMEM_SHARED((NPAD, 16), jnp.float32),
        pltpu.VMEM((MSG_CHUNK, 128), jnp.int32),
        pltpu.VMEM((MSG_CHUNK, 128), jnp.int32),
        pltpu.VMEM((NBUF, 128, 16), jnp.float32),
        pltpu.VMEM((ZCH, 16), jnp.float32),
        pltpu.SemaphoreType.DMA,
        pltpu.SemaphoreType.DMA,
        pltpu.SemaphoreType.DMA,
        pltpu.SemaphoreType.DMA,
        pltpu.SemaphoreType.DMA,
        pltpu.SemaphoreType.DMA,
        pltpu.SemaphoreType.DMA,
        pltpu.SemaphoreType.DMA,
        pltpu.SemaphoreType.DMA,
        pltpu.SemaphoreType.DMA,
        pltpu.SemaphoreType.DMA,
        pltpu.SemaphoreType.DMA,
    ],
)
def _sc_message(src_hbm, dst_hbm, y0_hbm, y1_hbm, out0_hbm, out1_hbm,
                acc, src_v, dst_v, rows, zbuf,
                gsem0, gsem1, gsem2, gsem3, gsem4, gsem5,
                ssem0, ssem1, ssem2, ssem3, ssem4, ssem5):
    cid = lax.axis_index("c")
    sid = lax.axis_index("s")
    gsems = (gsem0, gsem1, gsem2, gsem3, gsem4, gsem5)
    ssems = (ssem0, ssem1, ssem2, ssem3, ssem4, ssem5)

    def _run(y_hbm, out_hbm):
        zero16 = jnp.zeros((16,), jnp.float32)

        def _zb(i, c):
            zbuf[i, :] = zero16
            return c

        lax.fori_loop(0, ZCH, _zb, 0, unroll=8)
        row0 = sid * ROWS_PER_TILE

        def _za(k, c):
            pltpu.sync_copy(zbuf, acc.at[pl.ds(row0 + k * ZCH, ZCH)])
            return c

        lax.fori_loop(0, ROWS_PER_TILE // ZCH, _za, 0)
        plsc.subcore_barrier()

        blk0 = sid * MSG_BLOCKS_PER_TILE

        def _chunk(s, c):
            r0 = blk0 + s * MSG_CHUNK
            pltpu.sync_copy(src_hbm.at[pl.ds(r0, MSG_CHUNK)], src_v)
            pltpu.sync_copy(dst_hbm.at[pl.ds(r0, MSG_CHUNK)], dst_v)
            gd, sd = {}, {}
            for j in range(PIPE):
                gd[j % NBUF] = pltpu.async_copy(
                    y_hbm.at[src_v.at[j]], rows.at[j % NBUF], gsems[j % NBUF])
            for j in range(MSG_CHUNK):
                g = j + PIPE
                if g < MSG_CHUNK:
                    bg = g % NBUF
                    if g >= NBUF:
                        sd[bg].wait()
                    gd[bg] = pltpu.async_copy(
                        y_hbm.at[src_v.at[g]], rows.at[bg], gsems[bg])
                b = j % NBUF
                gd[b].wait()
                sd[b] = pltpu.async_copy(
                    rows.at[b], acc.at[dst_v.at[j]], ssems[b], add=True)
            for b in range(NBUF):
                sd[b].wait()
            return c

        lax.fori_loop(0, MSG_STAGES, _chunk, 0)
        plsc.subcore_barrier()
        pltpu.sync_copy(acc.at[pl.ds(row0, ROWS_PER_TILE)],
                        out_hbm.at[pl.ds(row0, ROWS_PER_TILE), pl.ds(0, 16)])

    @pl.when(cid == 0)
    def _():
        _run(y0_hbm, out0_hbm)

    @pl.when(cid == 1)
    def _():
        _run(y1_hbm, out1_hbm)


_BLK = 1024
_GRID = NPAD // _BLK


def _prep_body(x_ref, w1_ref, d0_ref, d1_ref, d2_ref, d3_ref,
               y0_ref, y1_ref, dinv_ref):
    deg = (jnp.sum(d0_ref[...], axis=0) + jnp.sum(d1_ref[...], axis=0)
           + jnp.sum(d2_ref[...], axis=0) + jnp.sum(d3_ref[...], axis=0))
    dinv = lax.rsqrt(deg)
    xw = jnp.dot(x_ref[...], w1_ref[...], preferred_element_type=jnp.float32)
    y = xw * dinv[:, None]
    zpad = jnp.zeros((_BLK, 128 - 16), jnp.float32)
    y0_ref[...] = jnp.concatenate([y[:, :16], zpad], axis=1)
    y1_ref[...] = jnp.concatenate([y[:, 16:], zpad], axis=1)
    dinv_ref[...] = dinv[None, :]


_prep = pl.pallas_call(
    _prep_body,
    grid=(_GRID,),
    in_specs=[
        pl.BlockSpec((_BLK, F_IN), lambda i: (i, 0)),
        pl.BlockSpec((F_IN, H), lambda i: (0, 0)),
        pl.BlockSpec((8, _BLK), lambda i: (0, i)),
        pl.BlockSpec((8, _BLK), lambda i: (0, i)),
        pl.BlockSpec((8, _BLK), lambda i: (0, i)),
        pl.BlockSpec((8, _BLK), lambda i: (0, i)),
    ],
    out_specs=[
        pl.BlockSpec((_BLK, 128), lambda i: (i, 0)),
        pl.BlockSpec((_BLK, 128), lambda i: (i, 0)),
        pl.BlockSpec((1, _BLK), lambda i: (0, i)),
    ],
    out_shape=[
        jax.ShapeDtypeStruct((NPAD, 128), jnp.float32),
        jax.ShapeDtypeStruct((NPAD, 128), jnp.float32),
        jax.ShapeDtypeStruct((1, NPAD), jnp.float32),
    ],
)


def _head_body(acc0_ref, acc1_ref, dinv_ref,
               b1_ref, w2_ref, b2_ref, w3_ref, b3_ref, w4_ref, b4_ref,
               out_ref):
    acc = jnp.concatenate([acc0_ref[:, :16], acc1_ref[:, :16]], axis=1)
    dinv = jnp.transpose(dinv_ref[...], (1, 0))
    h = jnp.maximum(acc * dinv + b1_ref[...], 0.0)
    h = jnp.maximum(
        jnp.dot(h, w2_ref[...], preferred_element_type=jnp.float32)
        + b2_ref[...], 0.0)
    h = jnp.maximum(
        jnp.dot(h, w3_ref[...], preferred_element_type=jnp.float32)
        + b3_ref[...], 0.0)
    logits = (jnp.dot(h, w4_ref[...], preferred_element_type=jnp.float32)
              + b4_ref[...])
    m = jnp.max(logits, axis=1, keepdims=True)
    lse = jnp.log(jnp.sum(jnp.exp(logits - m), axis=1, keepdims=True)) + m
    out_ref[...] = logits - lse


_head = pl.pallas_call(
    _head_body,
    grid=(_GRID,),
    in_specs=[
        pl.BlockSpec((_BLK, 128), lambda i: (i, 0)),
        pl.BlockSpec((_BLK, 128), lambda i: (i, 0)),
        pl.BlockSpec((1, _BLK), lambda i: (0, i)),
        pl.BlockSpec((1, H), lambda i: (0, 0)),
        pl.BlockSpec((H, H // 2), lambda i: (0, 0)),
        pl.BlockSpec((1, H // 2), lambda i: (0, 0)),
        pl.BlockSpec((H // 2, H // 4), lambda i: (0, 0)),
        pl.BlockSpec((1, H // 4), lambda i: (0, 0)),
        pl.BlockSpec((H // 4, C), lambda i: (0, 0)),
        pl.BlockSpec((1, C), lambda i: (0, 0)),
    ],
    out_specs=pl.BlockSpec((_BLK, C), lambda i: (i, 0)),
    out_shape=jax.ShapeDtypeStruct((N, C), jnp.float32),
)


def kernel(x, edge_index, edge_weight, W1, b1, W2, b2, W3, b3, W4, b4):
    del edge_weight
    e3 = edge_index.reshape(2, ER_IN, 128)
    srcp, dstp = _edges(e3, e3)

    d0, d1, d2, d3 = _sc_degree(dstp)
    y0p, y1p, dinvr = _prep(x, W1, d0, d1, d2, d3)
    acc0, acc1 = _sc_message(srcp, dstp,
                             y0p.reshape(NPAD * 8, 16),
                             y1p.reshape(NPAD * 8, 16))
    return _head(acc0, acc1, dinvr,
                 b1.reshape(1, H), W2, b2.reshape(1, H // 2),
                 W3, b3.reshape(1, H // 4), W4, b4.reshape(1, C))


# speedup vs baseline: 53.7088x; 1.0523x over previous
import functools

import jax
import jax.numpy as jnp
from jax import lax
from jax.experimental import pallas as pl
from jax.experimental.pallas import tpu as pltpu
from jax.experimental.pallas import tpu_sc as plsc

N = 100000
E = 1600000
F_IN = 128
H = 32
C = 10

NC = 2
NS = 16

NPAD = 100352
ESELF = E + NPAD
EPAD = 1703936
EROWS = EPAD // 128
ER_IN = E // 128

ROWS_PER_TILE = NPAD // NS
ZCH = 224

DEG_ROWS_PER_W = EROWS // (NC * NS)
DEG_CHUNK = 52
DEG_STAGES = DEG_ROWS_PER_W // DEG_CHUNK

MSG_BLOCKS_PER_TILE = EROWS // NS
MSG_CHUNK = 32
MSG_STAGES = MSG_BLOCKS_PER_TILE // MSG_CHUNK
NBUF = 8
PIPE = 4

_mesh = plsc.VectorSubcoreMesh(core_axis_name="c", subcore_axis_name="s")


def _edges_body(src_in, dst_in, srcp_ref, dstp_ref):
    i = pl.program_id(0)
    row = i * 256 + lax.broadcasted_iota(jnp.int32, (256, 128), 0)
    lane = lax.broadcasted_iota(jnp.int32, (256, 128), 1)
    self_id = (row - ER_IN) * 128 + lane
    tail = jnp.where(row < ER_IN + NPAD // 128, self_id, N)
    srcp_ref[...] = jnp.where(row < ER_IN, src_in[0], tail) * 8
    dstp_ref[...] = jnp.where(row < ER_IN, dst_in[0], tail)


_edges = pl.pallas_call(
    _edges_body,
    grid=(EROWS // 256,),
    in_specs=[
        pl.BlockSpec((1, 256, 128),
                     lambda i: (0, jnp.minimum(i, ER_IN // 256), 0)),
        pl.BlockSpec((1, 256, 128),
                     lambda i: (1, jnp.minimum(i, ER_IN // 256), 0)),
    ],
    out_specs=[
        pl.BlockSpec((256, 128), lambda i: (i, 0)),
        pl.BlockSpec((256, 128), lambda i: (i, 0)),
    ],
    out_shape=[
        jax.ShapeDtypeStruct((EROWS, 128), jnp.int32),
        jax.ShapeDtypeStruct((EROWS, 128), jnp.int32),
    ],
)


@functools.partial(
    pl.kernel,
    out_type=tuple(
        jax.ShapeDtypeStruct((8, NPAD), jnp.float32) for _ in range(4)),
    mesh=_mesh,
    compiler_params=pltpu.CompilerParams(
        needs_layout_passes=False, use_tc_tiling_on_sc=False),
    scratch_types=[
        pltpu.VMEM((NPAD,), jnp.float32),
        pltpu.VMEM((DEG_CHUNK, 128), jnp.int32),
    ],
)
def _sc_degree(dst_hbm, out0, out1, out2, out3, deg_v, idx_v):
    cid = lax.axis_index("c")
    sid = lax.axis_index("s")
    wid = sid * NC + cid
    zero16 = jnp.zeros((16,), jnp.float32)
    ones16 = jnp.ones((16,), jnp.float32)

    def _zero(i, c):
        deg_v[pl.ds(i * 16, 16)] = zero16
        return c

    lax.fori_loop(0, NPAD // 16, _zero, 0, unroll=8)

    base = wid * DEG_ROWS_PER_W

    def _stage(s, c):
        pltpu.sync_copy(dst_hbm.at[pl.ds(base + s * DEG_CHUNK, DEG_CHUNK)],
                        idx_v)

        def _row(j, c2):
            for k in range(8):
                idx = idx_v[j, pl.ds(k * 16, 16)]
                plsc.addupdate_scatter(deg_v, [idx], ones16)
            return c2

        lax.fori_loop(0, DEG_CHUNK, _row, 0, unroll=2)
        return c

    lax.fori_loop(0, DEG_STAGES, _stage, 0)
    for k, out_k in enumerate((out0, out1, out2, out3)):
        @pl.when(wid // 8 == k)
        def _(out_k=out_k):
            pltpu.sync_copy(deg_v, out_k.at[wid % 8])


@functools.partial(
    pl.kernel,
    out_type=(
        jax.ShapeDtypeStruct((NPAD, 128), jnp.float32),
        jax.ShapeDtypeStruct((NPAD, 128), jnp.float32),
    ),
    mesh=_mesh,
    compiler_params=pltpu.CompilerParams(
        needs_layout_passes=False, use_tc_tiling_on_sc=False),
    scratch_types=[
        pltpu.VMEM_SHARED((NPAD, 16), jnp.float32),
        pltpu.VMEM((MSG_CHUNK, 128), jnp.int32),
        pltpu.VMEM((MSG_CHUNK, 128), jnp.int32),
        pltpu.VMEM((NBUF, 128, 16), jnp.float32),
        pltpu.VMEM((ZCH, 16), jnp.float32),
    ] + [pltpu.SemaphoreType.DMA] * 16,
)
def _sc_message(src_hbm, dst_hbm, y0_hbm, y1_hbm, out0_hbm, out1_hbm,
                acc, src_v, dst_v, rows, zbuf, *sems):
    cid = lax.axis_index("c")
    sid = lax.axis_index("s")
    gsems = sems[:NBUF]
    ssems = sems[NBUF:]

    def _run(y_hbm, out_hbm):
        zero16 = jnp.zeros((16,), jnp.float32)

        def _zb(i, c):
            zbuf[i, :] = zero16
            return c

        lax.fori_loop(0, ZCH, _zb, 0, unroll=8)
        row0 = sid * ROWS_PER_TILE

        def _za(k, c):
            pltpu.sync_copy(zbuf, acc.at[pl.ds(row0 + k * ZCH, ZCH)])
            return c

        lax.fori_loop(0, ROWS_PER_TILE // ZCH, _za, 0)
        plsc.subcore_barrier()

        blk0 = sid * MSG_BLOCKS_PER_TILE

        def _chunk(s, c):
            r0 = blk0 + s * MSG_CHUNK
            pltpu.sync_copy(src_hbm.at[pl.ds(r0, MSG_CHUNK)], src_v)
            pltpu.sync_copy(dst_hbm.at[pl.ds(r0, MSG_CHUNK)], dst_v)
            gd, sd = {}, {}
            for j in range(PIPE):
                gd[j % NBUF] = pltpu.async_copy(
                    y_hbm.at[src_v.at[j]], rows.at[j % NBUF], gsems[j % NBUF])
            for j in range(MSG_CHUNK):
                g = j + PIPE
                if g < MSG_CHUNK:
                    bg = g % NBUF
                    if g >= NBUF:
                        sd[bg].wait()
                    gd[bg] = pltpu.async_copy(
                        y_hbm.at[src_v.at[g]], rows.at[bg], gsems[bg])
                b = j % NBUF
                gd[b].wait()
                sd[b] = pltpu.async_copy(
                    rows.at[b], acc.at[dst_v.at[j]], ssems[b], add=True)
            for b in range(NBUF):
                sd[b].wait()
            return c

        lax.fori_loop(0, MSG_STAGES, _chunk, 0)
        plsc.subcore_barrier()
        pltpu.sync_copy(acc.at[pl.ds(row0, ROWS_PER_TILE)],
                        out_hbm.at[pl.ds(row0, ROWS_PER_TILE), pl.ds(0, 16)])

    @pl.when(cid == 0)
    def _():
        _run(y0_hbm, out0_hbm)

    @pl.when(cid == 1)
    def _():
        _run(y1_hbm, out1_hbm)


_BLK = 1024
_GRID = NPAD // _BLK


def _prep_body(x_ref, w1_ref, d0_ref, d1_ref, d2_ref, d3_ref,
               y0_ref, y1_ref, dinv_ref):
    deg = (jnp.sum(d0_ref[...], axis=0) + jnp.sum(d1_ref[...], axis=0)
           + jnp.sum(d2_ref[...], axis=0) + jnp.sum(d3_ref[...], axis=0))
    dinv = lax.rsqrt(deg)
    xf = x_ref[...]
    wf = w1_ref[...]
    xh = xf.astype(jnp.bfloat16)
    xl = (xf - xh.astype(jnp.float32)).astype(jnp.bfloat16)
    wh = wf.astype(jnp.bfloat16)
    wl = (wf - wh.astype(jnp.float32)).astype(jnp.bfloat16)
    xw = (jnp.dot(xh, wh, preferred_element_type=jnp.float32)
          + jnp.dot(xl, wh, preferred_element_type=jnp.float32)
          + jnp.dot(xh, wl, preferred_element_type=jnp.float32))
    y = xw * dinv[:, None]
    zpad = jnp.zeros((_BLK, 128 - 16), jnp.float32)
    y0_ref[...] = jnp.concatenate([y[:, :16], zpad], axis=1)
    y1_ref[...] = jnp.concatenate([y[:, 16:], zpad], axis=1)
    dinv_ref[...] = dinv[None, :]


_prep = pl.pallas_call(
    _prep_body,
    grid=(_GRID,),
    in_specs=[
        pl.BlockSpec((_BLK, F_IN), lambda i: (i, 0)),
        pl.BlockSpec((F_IN, H), lambda i: (0, 0)),
        pl.BlockSpec((8, _BLK), lambda i: (0, i)),
        pl.BlockSpec((8, _BLK), lambda i: (0, i)),
        pl.BlockSpec((8, _BLK), lambda i: (0, i)),
        pl.BlockSpec((8, _BLK), lambda i: (0, i)),
    ],
    out_specs=[
        pl.BlockSpec((_BLK, 128), lambda i: (i, 0)),
        pl.BlockSpec((_BLK, 128), lambda i: (i, 0)),
        pl.BlockSpec((1, _BLK), lambda i: (0, i)),
    ],
    out_shape=[
        jax.ShapeDtypeStruct((NPAD, 128), jnp.float32),
        jax.ShapeDtypeStruct((NPAD, 128), jnp.float32),
        jax.ShapeDtypeStruct((1, NPAD), jnp.float32),
    ],
)


def _head_body(acc0_ref, acc1_ref, dinv_ref,
               b1_ref, w2_ref, b2_ref, w3_ref, b3_ref, w4_ref, b4_ref,
               out_ref):
    acc = jnp.concatenate([acc0_ref[:, :16], acc1_ref[:, :16]], axis=1)
    dinv = jnp.transpose(dinv_ref[...], (1, 0))
    h = jnp.maximum(acc * dinv + b1_ref[...], 0.0)
    h = jnp.maximum(
        jnp.dot(h, w2_ref[...], preferred_element_type=jnp.float32)
        + b2_ref[...], 0.0)
    h = jnp.maximum(
        jnp.dot(h, w3_ref[...], preferred_element_type=jnp.float32)
        + b3_ref[...], 0.0)
    logits = (jnp.dot(h, w4_ref[...], preferred_element_type=jnp.float32)
              + b4_ref[...])
    m = jnp.max(logits, axis=1, keepdims=True)
    lse = jnp.log(jnp.sum(jnp.exp(logits - m), axis=1, keepdims=True)) + m
    out_ref[...] = logits - lse


_head = pl.pallas_call(
    _head_body,
    grid=(_GRID,),
    in_specs=[
        pl.BlockSpec((_BLK, 128), lambda i: (i, 0)),
        pl.BlockSpec((_BLK, 128), lambda i: (i, 0)),
        pl.BlockSpec((1, _BLK), lambda i: (0, i)),
        pl.BlockSpec((1, H), lambda i: (0, 0)),
        pl.BlockSpec((H, H // 2), lambda i: (0, 0)),
        pl.BlockSpec((1, H // 2), lambda i: (0, 0)),
        pl.BlockSpec((H // 2, H // 4), lambda i: (0, 0)),
        pl.BlockSpec((1, H // 4), lambda i: (0, 0)),
        pl.BlockSpec((H // 4, C), lambda i: (0, 0)),
        pl.BlockSpec((1, C), lambda i: (0, 0)),
    ],
    out_specs=pl.BlockSpec((_BLK, C), lambda i: (i, 0)),
    out_shape=jax.ShapeDtypeStruct((N, C), jnp.float32),
)


def kernel(x, edge_index, edge_weight, W1, b1, W2, b2, W3, b3, W4, b4):
    del edge_weight
    e3 = edge_index.reshape(2, ER_IN, 128)
    srcp, dstp = _edges(e3, e3)

    d0, d1, d2, d3 = _sc_degree(dstp)
    y0p, y1p, dinvr = _prep(x, W1, d0, d1, d2, d3)
    acc0, acc1 = _sc_message(srcp, dstp,
                             y0p.reshape(NPAD * 8, 16),
                             y1p.reshape(NPAD * 8, 16))
    return _head(acc0, acc1, dinvr,
                 b1.reshape(1, H), W2, b2.reshape(1, H // 2),
                 W3, b3.reshape(1, H // 4), W4, b4.reshape(1, C))


# speedup vs baseline: 57.4296x; 1.0693x over previous
import functools

import jax
import jax.numpy as jnp
from jax import lax
from jax.experimental import pallas as pl
from jax.experimental.pallas import tpu as pltpu
from jax.experimental.pallas import tpu_sc as plsc

N = 100000
E = 1600000
F_IN = 128
H = 32
C = 10

NC = 2
NS = 16

NPAD = 100352
ESELF = E + NPAD
EPAD = 1703936
EROWS = EPAD // 128
ER_IN = E // 128

ROWS_PER_TILE = NPAD // NS
ZCH = 224

DEG_ROWS_PER_W = EROWS // (NC * NS)
DEG_CHUNK = 52
DEG_STAGES = DEG_ROWS_PER_W // DEG_CHUNK

MSG_BLOCKS_PER_TILE = EROWS // NS
MSG_CHUNK = 32
MSG_STAGES = MSG_BLOCKS_PER_TILE // MSG_CHUNK
NBUF = 8
PIPE = 4

_mesh = plsc.VectorSubcoreMesh(core_axis_name="c", subcore_axis_name="s")


def _edges_body(src_in, dst_in, srcp_ref, dstp_ref):
    i = pl.program_id(0)
    row = i * 256 + lax.broadcasted_iota(jnp.int32, (256, 128), 0)
    lane = lax.broadcasted_iota(jnp.int32, (256, 128), 1)
    self_id = (row - ER_IN) * 128 + lane
    tail = jnp.where(row < ER_IN + NPAD // 128, self_id, N)
    srcp_ref[...] = jnp.where(row < ER_IN, src_in[0], tail) * 8
    dstp_ref[...] = jnp.where(row < ER_IN, dst_in[0], tail)


_edges = pl.pallas_call(
    _edges_body,
    grid=(EROWS // 256,),
    in_specs=[
        pl.BlockSpec((1, 256, 128),
                     lambda i: (0, jnp.minimum(i, ER_IN // 256), 0)),
        pl.BlockSpec((1, 256, 128),
                     lambda i: (1, jnp.minimum(i, ER_IN // 256), 0)),
    ],
    out_specs=[
        pl.BlockSpec((256, 128), lambda i: (i, 0)),
        pl.BlockSpec((256, 128), lambda i: (i, 0)),
    ],
    out_shape=[
        jax.ShapeDtypeStruct((EROWS, 128), jnp.int32),
        jax.ShapeDtypeStruct((EROWS, 128), jnp.int32),
    ],
)


@functools.partial(
    pl.kernel,
    out_type=tuple(
        jax.ShapeDtypeStruct((8, NPAD), jnp.float32) for _ in range(4)),
    mesh=_mesh,
    compiler_params=pltpu.CompilerParams(
        needs_layout_passes=False, use_tc_tiling_on_sc=False),
    scratch_types=[
        pltpu.VMEM((NPAD,), jnp.float32),
        pltpu.VMEM((DEG_CHUNK, 128), jnp.int32),
    ],
)
def _sc_degree(dst_hbm, out0, out1, out2, out3, deg_v, idx_v):
    cid = lax.axis_index("c")
    sid = lax.axis_index("s")
    wid = sid * NC + cid
    zero16 = jnp.zeros((16,), jnp.float32)
    ones16 = jnp.ones((16,), jnp.float32)

    def _zero(i, c):
        deg_v[pl.ds(i * 16, 16)] = zero16
        return c

    lax.fori_loop(0, NPAD // 16, _zero, 0, unroll=8)

    base = wid * DEG_ROWS_PER_W

    def _stage(s, c):
        pltpu.sync_copy(dst_hbm.at[pl.ds(base + s * DEG_CHUNK, DEG_CHUNK)],
                        idx_v)

        def _row(j, c2):
            for k in range(8):
                idx = idx_v[j, pl.ds(k * 16, 16)]
                plsc.addupdate_scatter(deg_v, [idx], ones16)
            return c2

        lax.fori_loop(0, DEG_CHUNK, _row, 0, unroll=2)
        return c

    lax.fori_loop(0, DEG_STAGES, _stage, 0)
    for k, out_k in enumerate((out0, out1, out2, out3)):
        @pl.when(wid // 8 == k)
        def _(out_k=out_k):
            pltpu.sync_copy(deg_v, out_k.at[wid % 8])


@functools.partial(
    pl.kernel,
    out_type=(
        jax.ShapeDtypeStruct((NPAD, 16), jnp.float32),
        jax.ShapeDtypeStruct((NPAD, 16), jnp.float32),
    ),
    mesh=_mesh,
    compiler_params=pltpu.CompilerParams(
        needs_layout_passes=False, use_tc_tiling_on_sc=False),
    scratch_types=[
        pltpu.VMEM_SHARED((NPAD, 16), jnp.float32),
        pltpu.VMEM((MSG_CHUNK, 128), jnp.int32),
        pltpu.VMEM((MSG_CHUNK, 128), jnp.int32),
        pltpu.VMEM((NBUF, 128, 16), jnp.float32),
        pltpu.VMEM((ZCH, 16), jnp.float32),
    ] + [pltpu.SemaphoreType.DMA] * 16,
)
def _sc_message(src_hbm, dst_hbm, y0_hbm, y1_hbm, out0_hbm, out1_hbm,
                acc, src_v, dst_v, rows, zbuf, *sems):
    cid = lax.axis_index("c")
    sid = lax.axis_index("s")
    gsems = sems[:NBUF]
    ssems = sems[NBUF:]

    def _run(y_hbm, out_hbm):
        zero16 = jnp.zeros((16,), jnp.float32)

        def _zb(i, c):
            zbuf[i, :] = zero16
            return c

        lax.fori_loop(0, ZCH, _zb, 0, unroll=8)
        row0 = sid * ROWS_PER_TILE

        def _za(k, c):
            pltpu.sync_copy(zbuf, acc.at[pl.ds(row0 + k * ZCH, ZCH)])
            return c

        lax.fori_loop(0, ROWS_PER_TILE // ZCH, _za, 0)
        plsc.subcore_barrier()

        blk0 = sid * MSG_BLOCKS_PER_TILE

        def _chunk(s, c):
            r0 = blk0 + s * MSG_CHUNK
            pltpu.sync_copy(src_hbm.at[pl.ds(r0, MSG_CHUNK)], src_v)
            pltpu.sync_copy(dst_hbm.at[pl.ds(r0, MSG_CHUNK)], dst_v)
            gd, sd = {}, {}
            for j in range(PIPE):
                gd[j % NBUF] = pltpu.async_copy(
                    y_hbm.at[src_v.at[j]], rows.at[j % NBUF], gsems[j % NBUF])
            for j in range(MSG_CHUNK):
                g = j + PIPE
                if g < MSG_CHUNK:
                    bg = g % NBUF
                    if g >= NBUF:
                        sd[bg].wait()
                    gd[bg] = pltpu.async_copy(
                        y_hbm.at[src_v.at[g]], rows.at[bg], gsems[bg])
                b = j % NBUF
                gd[b].wait()
                sd[b] = pltpu.async_copy(
                    rows.at[b], acc.at[dst_v.at[j]], ssems[b], add=True)
            for b in range(NBUF):
                sd[b].wait()
            return c

        lax.fori_loop(0, MSG_STAGES, _chunk, 0)
        plsc.subcore_barrier()
        pltpu.sync_copy(acc.at[pl.ds(row0, ROWS_PER_TILE)],
                        out_hbm.at[pl.ds(row0, ROWS_PER_TILE)])

    @pl.when(cid == 0)
    def _():
        _run(y0_hbm, out0_hbm)

    @pl.when(cid == 1)
    def _():
        _run(y1_hbm, out1_hbm)


_BLK = 1024
_GRID = NPAD // _BLK


def _prep_body(x_ref, w1_ref, d0_ref, d1_ref, d2_ref, d3_ref,
               y0_ref, y1_ref, dinv_ref):
    deg = (jnp.sum(d0_ref[...], axis=0) + jnp.sum(d1_ref[...], axis=0)
           + jnp.sum(d2_ref[...], axis=0) + jnp.sum(d3_ref[...], axis=0))
    dinv = lax.rsqrt(deg)
    xf = x_ref[...]
    wf = w1_ref[...]
    xh = xf.astype(jnp.bfloat16)
    xl = (xf - xh.astype(jnp.float32)).astype(jnp.bfloat16)
    wh = wf.astype(jnp.bfloat16)
    wl = (wf - wh.astype(jnp.float32)).astype(jnp.bfloat16)
    xw = (jnp.dot(xh, wh, preferred_element_type=jnp.float32)
          + jnp.dot(xl, wh, preferred_element_type=jnp.float32)
          + jnp.dot(xh, wl, preferred_element_type=jnp.float32))
    y = xw * dinv[:, None]
    zpad = jnp.zeros((_BLK, 128 - 16), jnp.float32)
    y0_ref[...] = jnp.concatenate([y[:, :16], zpad], axis=1)
    y1_ref[...] = jnp.concatenate([y[:, 16:], zpad], axis=1)
    dinv_ref[...] = dinv[None, :]


_prep = pl.pallas_call(
    _prep_body,
    grid=(_GRID,),
    in_specs=[
        pl.BlockSpec((_BLK, F_IN), lambda i: (i, 0)),
        pl.BlockSpec((F_IN, H), lambda i: (0, 0)),
        pl.BlockSpec((8, _BLK), lambda i: (0, i)),
        pl.BlockSpec((8, _BLK), lambda i: (0, i)),
        pl.BlockSpec((8, _BLK), lambda i: (0, i)),
        pl.BlockSpec((8, _BLK), lambda i: (0, i)),
    ],
    out_specs=[
        pl.BlockSpec((_BLK, 128), lambda i: (i, 0)),
        pl.BlockSpec((_BLK, 128), lambda i: (i, 0)),
        pl.BlockSpec((1, _BLK), lambda i: (0, i)),
    ],
    out_shape=[
        jax.ShapeDtypeStruct((NPAD, 128), jnp.float32),
        jax.ShapeDtypeStruct((NPAD, 128), jnp.float32),
        jax.ShapeDtypeStruct((1, NPAD), jnp.float32),
    ],
)


def _head_body(acc0_ref, acc1_ref, dinv_ref,
               b1a_ref, b1b_ref, w2a_ref, w2b_ref, b2_ref,
               w3_ref, b3_ref, w4_ref, b4_ref, gs_ref, out_ref):
    dv = dinv_ref[...]
    g0 = jnp.maximum(acc0_ref[...] * dv + b1a_ref[...], 0.0)
    g1 = jnp.maximum(acc1_ref[...] * dv + b1b_ref[...], 0.0)
    h2 = jnp.maximum(
        jnp.dot(g0, w2a_ref[...], preferred_element_type=jnp.float32)
        + jnp.dot(g1, w2b_ref[...], preferred_element_type=jnp.float32)
        + b2_ref[...], 0.0)
    h3 = jnp.maximum(
        jnp.dot(h2, w3_ref[...], preferred_element_type=jnp.float32)
        + b3_ref[...], 0.0)
    logv = (jnp.dot(h3, w4_ref[...], preferred_element_type=jnp.float32)
            + b4_ref[...])
    m = jnp.max(logv, axis=1, keepdims=True)
    ex = jnp.exp(logv - m)
    sums = jnp.dot(ex, gs_ref[...], preferred_element_type=jnp.float32)
    out_ref[...] = logv - (jnp.log(sums) + m)


_head = pl.pallas_call(
    _head_body,
    grid=(_GRID,),
    in_specs=[
        pl.BlockSpec((128, 128), lambda i: (i, 0)),
        pl.BlockSpec((128, 128), lambda i: (i, 0)),
        pl.BlockSpec((128, 128), lambda i: (i, 0)),
        pl.BlockSpec((1, 128), lambda i: (0, 0)),
        pl.BlockSpec((1, 128), lambda i: (0, 0)),
        pl.BlockSpec((128, 128), lambda i: (0, 0)),
        pl.BlockSpec((128, 128), lambda i: (0, 0)),
        pl.BlockSpec((1, 128), lambda i: (0, 0)),
        pl.BlockSpec((128, 64), lambda i: (0, 0)),
        pl.BlockSpec((1, 64), lambda i: (0, 0)),
        pl.BlockSpec((64, 80), lambda i: (0, 0)),
        pl.BlockSpec((1, 80), lambda i: (0, 0)),
        pl.BlockSpec((80, 80), lambda i: (0, 0)),
    ],
    out_specs=pl.BlockSpec((128, 80), lambda i: (i, 0)),
    out_shape=jax.ShapeDtypeStruct((NPAD // 8, 80), jnp.float32),
)


def kernel(x, edge_index, edge_weight, W1, b1, W2, b2, W3, b3, W4, b4):
    del edge_weight
    e3 = edge_index.reshape(2, ER_IN, 128)
    srcp, dstp = _edges(e3, e3)

    d0, d1, d2, d3 = _sc_degree(dstp)
    y0p, y1p, dinvr = _prep(x, W1, d0, d1, d2, d3)
    acc0, acc1 = _sc_message(srcp, dstp,
                             y0p.reshape(NPAD * 8, 16),
                             y1p.reshape(NPAD * 8, 16))
    eye8 = jnp.eye(8, dtype=jnp.float32)
    dinvrep = jnp.broadcast_to(dinvr.reshape(NPAD, 1),
                               (NPAD, 16)).reshape(NPAD // 8, 128)
    outv = _head(
        acc0.reshape(NPAD // 8, 128), acc1.reshape(NPAD // 8, 128), dinvrep,
        jnp.tile(b1[:16], 8).reshape(1, 128),
        jnp.tile(b1[16:], 8).reshape(1, 128),
        jnp.kron(eye8, W2[:16]), jnp.kron(eye8, W2[16:]),
        jnp.tile(b2, 8).reshape(1, 128),
        jnp.kron(eye8, W3), jnp.tile(b3, 8).reshape(1, 64),
        jnp.kron(eye8, W4), jnp.tile(b4, 8).reshape(1, 80),
        jnp.kron(eye8, jnp.ones((10, 10), jnp.float32)))
    return outv.reshape(NPAD, 10)[:N]


# speedup vs baseline: 59.6867x; 1.0393x over previous
import functools

import jax
import jax.numpy as jnp
from jax import lax
from jax.experimental import pallas as pl
from jax.experimental.pallas import tpu as pltpu
from jax.experimental.pallas import tpu_sc as plsc

N = 100000
E = 1600000
F_IN = 128
H = 32
C = 10

NC = 2
NS = 16

NPAD = 100352
ESELF = E + NPAD
EPAD = 1703936
EROWS = EPAD // 128
ER_IN = E // 128

ROWS_PER_TILE = NPAD // NS
ZCH = 224

DEG_ROWS_PER_W = EROWS // (NC * NS)
DEG_CHUNK = 52
DEG_STAGES = DEG_ROWS_PER_W // DEG_CHUNK

MSG_BLOCKS_PER_TILE = EROWS // NS
MSG_CHUNK = 32
MSG_STAGES = MSG_BLOCKS_PER_TILE // MSG_CHUNK
NBUF = 8
PIPE = 6

_mesh = plsc.VectorSubcoreMesh(core_axis_name="c", subcore_axis_name="s")


def _edges_body(src_in, dst_in, srcp_ref, dstp_ref):
    i = pl.program_id(0)
    row = i * 256 + lax.broadcasted_iota(jnp.int32, (256, 128), 0)
    lane = lax.broadcasted_iota(jnp.int32, (256, 128), 1)
    self_id = (row - ER_IN) * 128 + lane
    tail = jnp.where(row < ER_IN + NPAD // 128, self_id, N)
    srcp_ref[...] = jnp.where(row < ER_IN, src_in[0], tail) * 8
    dstp_ref[...] = jnp.where(row < ER_IN, dst_in[0], tail)


_edges = pl.pallas_call(
    _edges_body,
    grid=(EROWS // 256,),
    in_specs=[
        pl.BlockSpec((1, 256, 128),
                     lambda i: (0, jnp.minimum(i, ER_IN // 256), 0)),
        pl.BlockSpec((1, 256, 128),
                     lambda i: (1, jnp.minimum(i, ER_IN // 256), 0)),
    ],
    out_specs=[
        pl.BlockSpec((256, 128), lambda i: (i, 0)),
        pl.BlockSpec((256, 128), lambda i: (i, 0)),
    ],
    out_shape=[
        jax.ShapeDtypeStruct((EROWS, 128), jnp.int32),
        jax.ShapeDtypeStruct((EROWS, 128), jnp.int32),
    ],
)


@functools.partial(
    pl.kernel,
    out_type=tuple(
        jax.ShapeDtypeStruct((8, NPAD), jnp.float32) for _ in range(4)),
    mesh=_mesh,
    compiler_params=pltpu.CompilerParams(
        needs_layout_passes=False, use_tc_tiling_on_sc=False),
    scratch_types=[
        pltpu.VMEM((NPAD,), jnp.float32),
        pltpu.VMEM((DEG_CHUNK, 128), jnp.int32),
    ],
)
def _sc_degree(dst_hbm, out0, out1, out2, out3, deg_v, idx_v):
    cid = lax.axis_index("c")
    sid = lax.axis_index("s")
    wid = sid * NC + cid
    zero16 = jnp.zeros((16,), jnp.float32)
    ones16 = jnp.ones((16,), jnp.float32)

    def _zero(i, c):
        deg_v[pl.ds(i * 16, 16)] = zero16
        return c

    lax.fori_loop(0, NPAD // 16, _zero, 0, unroll=8)

    base = wid * DEG_ROWS_PER_W

    def _stage(s, c):
        pltpu.sync_copy(dst_hbm.at[pl.ds(base + s * DEG_CHUNK, DEG_CHUNK)],
                        idx_v)

        def _row(j, c2):
            for k in range(8):
                idx = idx_v[j, pl.ds(k * 16, 16)]
                plsc.addupdate_scatter(deg_v, [idx], ones16)
            return c2

        lax.fori_loop(0, DEG_CHUNK, _row, 0, unroll=2)
        return c

    lax.fori_loop(0, DEG_STAGES, _stage, 0)
    for k, out_k in enumerate((out0, out1, out2, out3)):
        @pl.when(wid // 8 == k)
        def _(out_k=out_k):
            pltpu.sync_copy(deg_v, out_k.at[wid % 8])


@functools.partial(
    pl.kernel,
    out_type=(
        jax.ShapeDtypeStruct((NPAD, 16), jnp.float32),
        jax.ShapeDtypeStruct((NPAD, 16), jnp.float32),
    ),
    mesh=_mesh,
    compiler_params=pltpu.CompilerParams(
        needs_layout_passes=False, use_tc_tiling_on_sc=False),
    scratch_types=[
        pltpu.VMEM_SHARED((NPAD, 16), jnp.float32),
        pltpu.VMEM((MSG_CHUNK, 128), jnp.int32),
        pltpu.VMEM((MSG_CHUNK, 128), jnp.int32),
        pltpu.VMEM((NBUF, 128, 16), jnp.float32),
        pltpu.VMEM((ZCH, 16), jnp.float32),
    ] + [pltpu.SemaphoreType.DMA] * 16,
)
def _sc_message(src_hbm, dst_hbm, y0_hbm, y1_hbm, out0_hbm, out1_hbm,
                acc, src_v, dst_v, rows, zbuf, *sems):
    cid = lax.axis_index("c")
    sid = lax.axis_index("s")
    gsems = sems[:NBUF]
    ssems = sems[NBUF:]

    def _run(y_hbm, out_hbm):
        zero16 = jnp.zeros((16,), jnp.float32)

        def _zb(i, c):
            zbuf[i, :] = zero16
            return c

        lax.fori_loop(0, ZCH, _zb, 0, unroll=8)
        row0 = sid * ROWS_PER_TILE

        def _za(k, c):
            pltpu.sync_copy(zbuf, acc.at[pl.ds(row0 + k * ZCH, ZCH)])
            return c

        lax.fori_loop(0, ROWS_PER_TILE // ZCH, _za, 0)
        plsc.subcore_barrier()

        blk0 = sid * MSG_BLOCKS_PER_TILE

        def _chunk(s, c):
            r0 = blk0 + s * MSG_CHUNK
            pltpu.sync_copy(src_hbm.at[pl.ds(r0, MSG_CHUNK)], src_v)
            pltpu.sync_copy(dst_hbm.at[pl.ds(r0, MSG_CHUNK)], dst_v)
            gd, sd = {}, {}
            for j in range(PIPE):
                gd[j % NBUF] = pltpu.async_copy(
                    y_hbm.at[src_v.at[j]], rows.at[j % NBUF], gsems[j % NBUF])
            for j in range(MSG_CHUNK):
                g = j + PIPE
                if g < MSG_CHUNK:
                    bg = g % NBUF
                    if g >= NBUF:
                        sd[bg].wait()
                    gd[bg] = pltpu.async_copy(
                        y_hbm.at[src_v.at[g]], rows.at[bg], gsems[bg])
                b = j % NBUF
                gd[b].wait()
                sd[b] = pltpu.async_copy(
                    rows.at[b], acc.at[dst_v.at[j]], ssems[b], add=True)
            for b in range(NBUF):
                sd[b].wait()
            return c

        lax.fori_loop(0, MSG_STAGES, _chunk, 0)
        plsc.subcore_barrier()
        pltpu.sync_copy(acc.at[pl.ds(row0, ROWS_PER_TILE)],
                        out_hbm.at[pl.ds(row0, ROWS_PER_TILE)])

    @pl.when(cid == 0)
    def _():
        _run(y0_hbm, out0_hbm)

    @pl.when(cid == 1)
    def _():
        _run(y1_hbm, out1_hbm)


_BLK = 1024
_GRID = NPAD // _BLK


def _prep_body(x_ref, w1_ref, d0_ref, d1_ref, d2_ref, d3_ref,
               y0_ref, y1_ref, dinv_ref):
    deg = (jnp.sum(d0_ref[...], axis=0) + jnp.sum(d1_ref[...], axis=0)
           + jnp.sum(d2_ref[...], axis=0) + jnp.sum(d3_ref[...], axis=0))
    dinv = lax.rsqrt(deg)
    xf = x_ref[...]
    wf = w1_ref[...]
    xh = xf.astype(jnp.bfloat16)
    xl = (xf - xh.astype(jnp.float32)).astype(jnp.bfloat16)
    wh = wf.astype(jnp.bfloat16)
    wl = (wf - wh.astype(jnp.float32)).astype(jnp.bfloat16)
    xw = (jnp.dot(xh, wh, preferred_element_type=jnp.float32)
          + jnp.dot(xl, wh, preferred_element_type=jnp.float32)
          + jnp.dot(xh, wl, preferred_element_type=jnp.float32))
    y = xw * dinv[:, None]
    zpad = jnp.zeros((_BLK, 128 - 16), jnp.float32)
    y0_ref[...] = jnp.concatenate([y[:, :16], zpad], axis=1)
    y1_ref[...] = jnp.concatenate([y[:, 16:], zpad], axis=1)
    dinv_ref[...] = dinv[None, :]


_prep = pl.pallas_call(
    _prep_body,
    grid=(_GRID,),
    in_specs=[
        pl.BlockSpec((_BLK, F_IN), lambda i: (i, 0)),
        pl.BlockSpec((F_IN, H), lambda i: (0, 0)),
        pl.BlockSpec((8, _BLK), lambda i: (0, i)),
        pl.BlockSpec((8, _BLK), lambda i: (0, i)),
        pl.BlockSpec((8, _BLK), lambda i: (0, i)),
        pl.BlockSpec((8, _BLK), lambda i: (0, i)),
    ],
    out_specs=[
        pl.BlockSpec((_BLK, 128), lambda i: (i, 0)),
        pl.BlockSpec((_BLK, 128), lambda i: (i, 0)),
        pl.BlockSpec((1, _BLK), lambda i: (0, i)),
    ],
    out_shape=[
        jax.ShapeDtypeStruct((NPAD, 128), jnp.float32),
        jax.ShapeDtypeStruct((NPAD, 128), jnp.float32),
        jax.ShapeDtypeStruct((1, NPAD), jnp.float32),
    ],
)


def _head_body(acc0_ref, acc1_ref, dinv_ref,
               b1a_ref, b1b_ref, w2a_ref, w2b_ref, b2_ref,
               w3_ref, b3_ref, w4_ref, b4_ref, gs_ref, out_ref):
    dv = dinv_ref[...]
    g0 = jnp.maximum(acc0_ref[...] * dv + b1a_ref[...], 0.0)
    g1 = jnp.maximum(acc1_ref[...] * dv + b1b_ref[...], 0.0)
    h2 = jnp.maximum(
        jnp.dot(g0, w2a_ref[...], preferred_element_type=jnp.float32)
        + jnp.dot(g1, w2b_ref[...], preferred_element_type=jnp.float32)
        + b2_ref[...], 0.0)
    h3 = jnp.maximum(
        jnp.dot(h2, w3_ref[...], preferred_element_type=jnp.float32)
        + b3_ref[...], 0.0)
    logv = (jnp.dot(h3, w4_ref[...], preferred_element_type=jnp.float32)
            + b4_ref[...])
    m = jnp.max(logv, axis=1, keepdims=True)
    ex = jnp.exp(logv - m)
    sums = jnp.dot(ex, gs_ref[...], preferred_element_type=jnp.float32)
    out_ref[...] = logv - (jnp.log(sums) + m)


_head = pl.pallas_call(
    _head_body,
    grid=(_GRID,),
    in_specs=[
        pl.BlockSpec((128, 128), lambda i: (i, 0)),
        pl.BlockSpec((128, 128), lambda i: (i, 0)),
        pl.BlockSpec((128, 128), lambda i: (i, 0)),
        pl.BlockSpec((1, 128), lambda i: (0, 0)),
        pl.BlockSpec((1, 128), lambda i: (0, 0)),
        pl.BlockSpec((128, 128), lambda i: (0, 0)),
        pl.BlockSpec((128, 128), lambda i: (0, 0)),
        pl.BlockSpec((1, 128), lambda i: (0, 0)),
        pl.BlockSpec((128, 64), lambda i: (0, 0)),
        pl.BlockSpec((1, 64), lambda i: (0, 0)),
        pl.BlockSpec((64, 80), lambda i: (0, 0)),
        pl.BlockSpec((1, 80), lambda i: (0, 0)),
        pl.BlockSpec((80, 80), lambda i: (0, 0)),
    ],
    out_specs=pl.BlockSpec((128, 80), lambda i: (i, 0)),
    out_shape=jax.ShapeDtypeStruct((NPAD // 8, 80), jnp.float32),
)


def kernel(x, edge_index, edge_weight, W1, b1, W2, b2, W3, b3, W4, b4):
    del edge_weight
    e3 = edge_index.reshape(2, ER_IN, 128)
    srcp, dstp = _edges(e3, e3)

    d0, d1, d2, d3 = _sc_degree(dstp)
    y0p, y1p, dinvr = _prep(x, W1, d0, d1, d2, d3)
    acc0, acc1 = _sc_message(srcp, dstp,
                             y0p.reshape(NPAD * 8, 16),
                             y1p.reshape(NPAD * 8, 16))
    eye8 = jnp.eye(8, dtype=jnp.float32)
    dinvrep = jnp.broadcast_to(dinvr.reshape(NPAD, 1),
                               (NPAD, 16)).reshape(NPAD // 8, 128)
    outv = _head(
        acc0.reshape(NPAD // 8, 128), acc1.reshape(NPAD // 8, 128), dinvrep,
        jnp.tile(b1[:16], 8).reshape(1, 128),
        jnp.tile(b1[16:], 8).reshape(1, 128),
        jnp.kron(eye8, W2[:16]), jnp.kron(eye8, W2[16:]),
        jnp.tile(b2, 8).reshape(1, 128),
        jnp.kron(eye8, W3), jnp.tile(b3, 8).reshape(1, 64),
        jnp.kron(eye8, W4), jnp.tile(b4, 8).reshape(1, 80),
        jnp.kron(eye8, jnp.ones((10, 10), jnp.float32)))
    return outv.reshape(NPAD, 10)[:N]


# speedup vs baseline: 59.9092x; 1.0037x over previous
import functools

import jax
import jax.numpy as jnp
from jax import lax
from jax.experimental import pallas as pl
from jax.experimental.pallas import tpu as pltpu
from jax.experimental.pallas import tpu_sc as plsc

N = 100000
E = 1600000
F_IN = 128
H = 32
C = 10

NC = 2
NS = 16

NPAD = 100352
ESELF = E + NPAD
EPAD = 1703936
EROWS = EPAD // 128
ER_IN = E // 128

ROWS_PER_TILE = NPAD // NS
ZCH = 224

DEG_ROWS_PER_W = EROWS // (NC * NS)
DEG_CHUNK = 52
DEG_STAGES = DEG_ROWS_PER_W // DEG_CHUNK

MSG_BLOCKS_PER_TILE = EROWS // NS
MSG_CHUNK = 32
MSG_STAGES = MSG_BLOCKS_PER_TILE // MSG_CHUNK
NBUF = 8
PIPE = 6

_mesh = plsc.VectorSubcoreMesh(core_axis_name="c", subcore_axis_name="s")


def _edges_tail(row, lane):
    self_id = (row - ER_IN) * 128 + lane
    return jnp.where(row < ER_IN + NPAD // 128, self_id, N)


def _edges_dst_body(dst_in, dstp_ref):
    i = pl.program_id(0)
    row = i * 256 + lax.broadcasted_iota(jnp.int32, (256, 128), 0)
    lane = lax.broadcasted_iota(jnp.int32, (256, 128), 1)
    dstp_ref[...] = jnp.where(row < ER_IN, dst_in[0], _edges_tail(row, lane))


def _edges_src_body(src_in, srcp_ref):
    i = pl.program_id(0)
    row = i * 256 + lax.broadcasted_iota(jnp.int32, (256, 128), 0)
    lane = lax.broadcasted_iota(jnp.int32, (256, 128), 1)
    srcp_ref[...] = jnp.where(row < ER_IN, src_in[0],
                              _edges_tail(row, lane)) * 8


def _make_edges(body, which):
    return pl.pallas_call(
        body,
        grid=(EROWS // 256,),
        in_specs=[
            pl.BlockSpec((1, 256, 128),
                         lambda i: (which, jnp.minimum(i, ER_IN // 256), 0)),
        ],
        out_specs=pl.BlockSpec((256, 128), lambda i: (i, 0)),
        out_shape=jax.ShapeDtypeStruct((EROWS, 128), jnp.int32),
    )


_edges_dst = _make_edges(_edges_dst_body, 1)
_edges_src = _make_edges(_edges_src_body, 0)


@functools.partial(
    pl.kernel,
    out_type=tuple(
        jax.ShapeDtypeStruct((8, NPAD), jnp.float32) for _ in range(4)),
    mesh=_mesh,
    compiler_params=pltpu.CompilerParams(
        needs_layout_passes=False, use_tc_tiling_on_sc=False),
    scratch_types=[
        pltpu.VMEM((NPAD,), jnp.float32),
        pltpu.VMEM((DEG_CHUNK, 128), jnp.int32),
    ],
)
def _sc_degree(dst_hbm, out0, out1, out2, out3, deg_v, idx_v):
    cid = lax.axis_index("c")
    sid = lax.axis_index("s")
    wid = sid * NC + cid
    zero16 = jnp.zeros((16,), jnp.float32)
    ones16 = jnp.ones((16,), jnp.float32)

    def _zero(i, c):
        deg_v[pl.ds(i * 16, 16)] = zero16
        return c

    lax.fori_loop(0, NPAD // 16, _zero, 0, unroll=8)

    base = wid * DEG_ROWS_PER_W

    def _stage(s, c):
        pltpu.sync_copy(dst_hbm.at[pl.ds(base + s * DEG_CHUNK, DEG_CHUNK)],
                        idx_v)

        def _row(j, c2):
            for k in range(8):
                idx = idx_v[j, pl.ds(k * 16, 16)]
                plsc.addupdate_scatter(deg_v, [idx], ones16)
            return c2

        lax.fori_loop(0, DEG_CHUNK, _row, 0, unroll=2)
        return c

    lax.fori_loop(0, DEG_STAGES, _stage, 0)
    for k, out_k in enumerate((out0, out1, out2, out3)):
        @pl.when(wid // 8 == k)
        def _(out_k=out_k):
            pltpu.sync_copy(deg_v, out_k.at[wid % 8])


@functools.partial(
    pl.kernel,
    out_type=(
        jax.ShapeDtypeStruct((NPAD, 16), jnp.float32),
        jax.ShapeDtypeStruct((NPAD, 16), jnp.float32),
    ),
    mesh=_mesh,
    compiler_params=pltpu.CompilerParams(
        needs_layout_passes=False, use_tc_tiling_on_sc=False),
    scratch_types=[
        pltpu.VMEM_SHARED((NPAD, 16), jnp.float32),
        pltpu.VMEM((MSG_CHUNK, 128), jnp.int32),
        pltpu.VMEM((MSG_CHUNK, 128), jnp.int32),
        pltpu.VMEM((NBUF, 128, 16), jnp.float32),
        pltpu.VMEM((ZCH, 16), jnp.float32),
    ] + [pltpu.SemaphoreType.DMA] * 16,
)
def _sc_message(src_hbm, dst_hbm, y0_hbm, y1_hbm, out0_hbm, out1_hbm,
                acc, src_v, dst_v, rows, zbuf, *sems):
    cid = lax.axis_index("c")
    sid = lax.axis_index("s")
    gsems = sems[:NBUF]
    ssems = sems[NBUF:]

    def _run(y_hbm, out_hbm):
        zero16 = jnp.zeros((16,), jnp.float32)

        def _zb(i, c):
            zbuf[i, :] = zero16
            return c

        lax.fori_loop(0, ZCH, _zb, 0, unroll=8)
        row0 = sid * ROWS_PER_TILE

        def _za(k, c):
            pltpu.sync_copy(zbuf, acc.at[pl.ds(row0 + k * ZCH, ZCH)])
            return c

        lax.fori_loop(0, ROWS_PER_TILE // ZCH, _za, 0)
        plsc.subcore_barrier()

        blk0 = sid * MSG_BLOCKS_PER_TILE

        def _chunk(s, c):
            r0 = blk0 + s * MSG_CHUNK
            pltpu.sync_copy(src_hbm.at[pl.ds(r0, MSG_CHUNK)], src_v)
            pltpu.sync_copy(dst_hbm.at[pl.ds(r0, MSG_CHUNK)], dst_v)
            gd, sd = {}, {}
            for j in range(PIPE):
                gd[j % NBUF] = pltpu.async_copy(
                    y_hbm.at[src_v.at[j]], rows.at[j % NBUF], gsems[j % NBUF])
            for j in range(MSG_CHUNK):
                g = j + PIPE
                if g < MSG_CHUNK:
                    bg = g % NBUF
                    if g >= NBUF:
                        sd[bg].wait()
                    gd[bg] = pltpu.async_copy(
                        y_hbm.at[src_v.at[g]], rows.at[bg], gsems[bg])
                b = j % NBUF
                gd[b].wait()
                sd[b] = pltpu.async_copy(
                    rows.at[b], acc.at[dst_v.at[j]], ssems[b], add=True)
            for b in range(NBUF):
                sd[b].wait()
            return c

        lax.fori_loop(0, MSG_STAGES, _chunk, 0)
        plsc.subcore_barrier()
        pltpu.sync_copy(acc.at[pl.ds(row0, ROWS_PER_TILE)],
                        out_hbm.at[pl.ds(row0, ROWS_PER_TILE)])

    @pl.when(cid == 0)
    def _():
        _run(y0_hbm, out0_hbm)

    @pl.when(cid == 1)
    def _():
        _run(y1_hbm, out1_hbm)


_BLK = 1024
_GRID = NPAD // _BLK


def _prep_body(x_ref, w1_ref, d0_ref, d1_ref, d2_ref, d3_ref,
               y0_ref, y1_ref, dinv_ref):
    deg = (jnp.sum(d0_ref[...], axis=0) + jnp.sum(d1_ref[...], axis=0)
           + jnp.sum(d2_ref[...], axis=0) + jnp.sum(d3_ref[...], axis=0))
    dinv = lax.rsqrt(deg)
    xf = x_ref[...]
    wf = w1_ref[...]
    xh = xf.astype(jnp.bfloat16)
    xl = (xf - xh.astype(jnp.float32)).astype(jnp.bfloat16)
    wh = wf.astype(jnp.bfloat16)
    wl = (wf - wh.astype(jnp.float32)).astype(jnp.bfloat16)
    xw = (jnp.dot(xh, wh, preferred_element_type=jnp.float32)
          + jnp.dot(xl, wh, preferred_element_type=jnp.float32)
          + jnp.dot(xh, wl, preferred_element_type=jnp.float32))
    y = xw * dinv[:, None]
    zpad = jnp.zeros((_BLK, 128 - 16), jnp.float32)
    y0_ref[...] = jnp.concatenate([y[:, :16], zpad], axis=1)
    y1_ref[...] = jnp.concatenate([y[:, 16:], zpad], axis=1)
    dinv_ref[...] = dinv[None, :]


_prep = pl.pallas_call(
    _prep_body,
    grid=(_GRID,),
    in_specs=[
        pl.BlockSpec((_BLK, F_IN), lambda i: (i, 0)),
        pl.BlockSpec((F_IN, H), lambda i: (0, 0)),
        pl.BlockSpec((8, _BLK), lambda i: (0, i)),
        pl.BlockSpec((8, _BLK), lambda i: (0, i)),
        pl.BlockSpec((8, _BLK), lambda i: (0, i)),
        pl.BlockSpec((8, _BLK), lambda i: (0, i)),
    ],
    out_specs=[
        pl.BlockSpec((_BLK, 128), lambda i: (i, 0)),
        pl.BlockSpec((_BLK, 128), lambda i: (i, 0)),
        pl.BlockSpec((1, _BLK), lambda i: (0, i)),
    ],
    out_shape=[
        jax.ShapeDtypeStruct((NPAD, 128), jnp.float32),
        jax.ShapeDtypeStruct((NPAD, 128), jnp.float32),
        jax.ShapeDtypeStruct((1, NPAD), jnp.float32),
    ],
)


def _head_body(acc0_ref, acc1_ref, dinv_ref,
               b1a_ref, b1b_ref, w2a_ref, w2b_ref, b2_ref,
               w3_ref, b3_ref, w4_ref, b4_ref, gs_ref, out_ref):
    dv = dinv_ref[...]
    g0 = jnp.maximum(acc0_ref[...] * dv + b1a_ref[...], 0.0)
    g1 = jnp.maximum(acc1_ref[...] * dv + b1b_ref[...], 0.0)
    h2 = jnp.maximum(
        jnp.dot(g0, w2a_ref[...], preferred_element_type=jnp.float32)
        + jnp.dot(g1, w2b_ref[...], preferred_element_type=jnp.float32)
        + b2_ref[...], 0.0)
    h3 = jnp.maximum(
        jnp.dot(h2, w3_ref[...], preferred_element_type=jnp.float32)
        + b3_ref[...], 0.0)
    logv = (jnp.dot(h3, w4_ref[...], preferred_element_type=jnp.float32)
            + b4_ref[...])
    m = jnp.max(logv, axis=1, keepdims=True)
    ex = jnp.exp(logv - m)
    sums = jnp.dot(ex, gs_ref[...], preferred_element_type=jnp.float32)
    out_ref[...] = logv - (jnp.log(sums) + m)


_head = pl.pallas_call(
    _head_body,
    grid=(_GRID,),
    in_specs=[
        pl.BlockSpec((128, 128), lambda i: (i, 0)),
        pl.BlockSpec((128, 128), lambda i: (i, 0)),
        pl.BlockSpec((128, 128), lambda i: (i, 0)),
        pl.BlockSpec((1, 128), lambda i: (0, 0)),
        pl.BlockSpec((1, 128), lambda i: (0, 0)),
        pl.BlockSpec((128, 128), lambda i: (0, 0)),
        pl.BlockSpec((128, 128), lambda i: (0, 0)),
        pl.BlockSpec((1, 128), lambda i: (0, 0)),
        pl.BlockSpec((128, 64), lambda i: (0, 0)),
        pl.BlockSpec((1, 64), lambda i: (0, 0)),
        pl.BlockSpec((64, 80), lambda i: (0, 0)),
        pl.BlockSpec((1, 80), lambda i: (0, 0)),
        pl.BlockSpec((80, 80), lambda i: (0, 0)),
    ],
    out_specs=pl.BlockSpec((128, 80), lambda i: (i, 0)),
    out_shape=jax.ShapeDtypeStruct((NPAD // 8, 80), jnp.float32),
)


def kernel(x, edge_index, edge_weight, W1, b1, W2, b2, W3, b3, W4, b4):
    del edge_weight
    e3 = edge_index.reshape(2, ER_IN, 128)
    dstp = _edges_dst(e3)
    d0, d1, d2, d3 = _sc_degree(dstp)
    srcp = _edges_src(e3)
    y0p, y1p, dinvr = _prep(x, W1, d0, d1, d2, d3)
    acc0, acc1 = _sc_message(srcp, dstp,
                             y0p.reshape(NPAD * 8, 16),
                             y1p.reshape(NPAD * 8, 16))
    eye8 = jnp.eye(8, dtype=jnp.float32)
    dinvrep = jnp.broadcast_to(dinvr.reshape(NPAD, 1),
                               (NPAD, 16)).reshape(NPAD // 8, 128)
    outv = _head(
        acc0.reshape(NPAD // 8, 128), acc1.reshape(NPAD // 8, 128), dinvrep,
        jnp.tile(b1[:16], 8).reshape(1, 128),
        jnp.tile(b1[16:], 8).reshape(1, 128),
        jnp.kron(eye8, W2[:16]), jnp.kron(eye8, W2[16:]),
        jnp.tile(b2, 8).reshape(1, 128),
        jnp.kron(eye8, W3), jnp.tile(b3, 8).reshape(1, 64),
        jnp.kron(eye8, W4), jnp.tile(b4, 8).reshape(1, 80),
        jnp.kron(eye8, jnp.ones((10, 10), jnp.float32)))
    return outv.reshape(NPAD, 10)[:N]
